# Initial kernel scaffold; baseline (speedup 1.0000x reference)
import jax
import jax.numpy as jnp
from jax.experimental import pallas as pl


def kernel(x, edge_index, batch, edge_attr, W1, a_src1, a_dst1, a_edge1, We1, b1, W2, a_src2, a_dst2, a_edge2, We2, b2, lin1_W, lin1_b, lin2_W, lin2_b):
    raise NotImplementedError("write your pallas kernel here")


# speedup vs baseline: 11.9304x; 11.9304x over previous
import functools

import jax
import jax.numpy as jnp
from jax import lax
from jax.experimental import pallas as pl
from jax.experimental.pallas import tpu as pltpu
from jax.experimental.pallas import tpu_sc as plsc

N = 36000
E = 576000
D_IN = 128
HID = 64
NG = 400
PG = 90

NC = 2
NS = 16
L = 16
NW = NC * NS

HW = HID // 2
AW = 48
XW = 48
DW = 16
EPT = E // NS
CH = 80
NCHUNK = EPT // CH
ROWS_PT = 2240
ROWS_TAIL = N - NS * ROWS_PT


def _edge_body(xl2_hbm, src_hbm, dst_hbm, ea_hbm, aldst2_hbm, c_hbm,
               zero_hbm, acc_out,
               c_v, src_v, dst_v, ea_v, idx_v, rows_v, adrows_v,
               buf2_v, acc_s, sem):
    cid = lax.axis_index("c")
    sid = lax.axis_index("s")

    pltpu.sync_copy(zero_hbm, acc_s.at[pl.ds(sid * ROWS_PT, ROWS_PT), :])
    @pl.when(sid == NS - 1)
    def _zero_tail():
        pltpu.sync_copy(zero_hbm.at[pl.ds(0, ROWS_TAIL), :],
                        acc_s.at[pl.ds(NS * ROWS_PT, ROWS_TAIL), :])

    pltpu.sync_copy(c_hbm, c_v)

    lanes = lax.iota(jnp.int32, L)
    zv = jnp.zeros((L,), jnp.float32)
    for i in range(CH):
        for j in range(AW // L):
            buf2_v[i, pl.ds(j * L, L)] = zv

    plsc.subcore_barrier()

    base = sid * EPT

    def chunk(k, carry):
        off = base + k * CH
        pltpu.sync_copy(src_hbm.at[pl.ds(off, CH)], src_v)
        pltpu.sync_copy(dst_hbm.at[pl.ds(off, CH)], dst_v)
        pltpu.sync_copy(ea_hbm.at[pl.ds(off, CH)], ea_v)
        cvec = c_v[pl.ds(0, L)]
        for g in range(CH // L):
            s16 = src_v[pl.ds(g * L, L)]
            idx_v[pl.ds(g * L, L)] = s16 * 2 + cid
        copy1 = pltpu.async_copy(xl2_hbm.at[idx_v], rows_v, sem)
        copy2 = pltpu.async_copy(aldst2_hbm.at[dst_v], adrows_v, sem)
        for g in range(CH // L):
            s16 = src_v[pl.ds(g * L, L)]
            d16 = dst_v[pl.ds(g * L, L)]
            e16 = ea_v[pl.ds(g * L, L)]
            validf = jnp.where(s16 != d16, 1.0, 0.0).astype(jnp.float32)
            rows16 = g * L + lanes
            plsc.store_scatter(buf2_v, [rows16, jnp.full((L,), 33, jnp.int32)], validf)
            plsc.store_scatter(buf2_v, [rows16, jnp.full((L,), 34, jnp.int32)], e16 * validf)
        copy1.wait()
        copy2.wait()
        c32 = jnp.full((L,), 32, jnp.int32)
        z16 = jnp.zeros((L,), jnp.int32)
        for g in range(CH // L):
            rows16 = g * L + lanes
            s16 = src_v[pl.ds(g * L, L)]
            d16 = dst_v[pl.ds(g * L, L)]
            e16 = ea_v[pl.ds(g * L, L)]
            asrc = plsc.load_gather(rows_v, [rows16, c32])
            adst = plsc.load_gather(adrows_v, [rows16, z16])
            alpha = asrc + adst + cvec * e16
            alpha = jnp.where(alpha >= 0, alpha, 0.2 * alpha)
            ex = jnp.exp(alpha) * jnp.where(s16 != d16, 1.0, 0.0)
            plsc.store_scatter(buf2_v, [rows16, c32], ex)
        def scale(i, c2):
            exs = buf2_v[i, pl.ds(2 * L, L)][0]
            buf2_v[i, pl.ds(0, L)] = rows_v[i, pl.ds(0, L)] * exs
            buf2_v[i, pl.ds(L, L)] = rows_v[i, pl.ds(L, L)] * exs
            return c2
        lax.fori_loop(0, CH, scale, 0, unroll=8)
        pltpu.sync_copy(buf2_v, acc_s.at[dst_v], add=True)
        return carry

    lax.fori_loop(0, NCHUNK, chunk, 0)
    plsc.subcore_barrier()
    pltpu.sync_copy(acc_s.at[pl.ds(sid * ROWS_PT, ROWS_PT), :],
                    acc_out.at[cid, pl.ds(sid * ROWS_PT, ROWS_PT), :])
    @pl.when(sid == NS - 1)
    def _flush_tail():
        pltpu.sync_copy(acc_s.at[pl.ds(NS * ROWS_PT, ROWS_TAIL), :],
                        acc_out.at[cid, pl.ds(NS * ROWS_PT, ROWS_TAIL), :])


_edge_kernel = functools.partial(
    pl.kernel,
    out_type=jax.ShapeDtypeStruct((NC, N, AW), jnp.float32),
    mesh=plsc.VectorSubcoreMesh(core_axis_name="c", subcore_axis_name="s",
                                num_cores=NC, num_subcores=NS),
    scratch_types=[
        pltpu.VMEM((L,), jnp.float32),
        pltpu.VMEM((CH,), jnp.int32),
        pltpu.VMEM((CH,), jnp.int32),
        pltpu.VMEM((CH,), jnp.float32),
        pltpu.VMEM((CH,), jnp.int32),
        pltpu.VMEM((CH, XW), jnp.float32),
        pltpu.VMEM((CH, DW), jnp.float32),
        pltpu.VMEM((CH, AW), jnp.float32),
        pltpu.VMEM_SHARED((N, AW), jnp.float32),
        pltpu.SemaphoreType.DMA,
    ],
    compiler_params=pltpu.CompilerParams(needs_layout_passes=False,
                                         use_tc_tiling_on_sc=False),
)(_edge_body)


_BM = 1440


def _xform_body(x_ref, w_ref, a2_ref, xl_ref, al2_ref):
    xl = jnp.dot(x_ref[...], w_ref[...], preferred_element_type=jnp.float32)
    xl_ref[...] = xl
    al2_ref[...] = jnp.dot(xl, a2_ref[...], preferred_element_type=jnp.float32)


def _xform(x, w, a_s, a_d):
    din = x.shape[1]
    a2 = jnp.stack([a_s, a_d], axis=1)
    return pl.pallas_call(
        _xform_body,
        grid=(N // _BM,),
        in_specs=[
            pl.BlockSpec((_BM, din), lambda i: (i, 0)),
            pl.BlockSpec((din, HID), lambda i: (0, 0)),
            pl.BlockSpec((HID, 2), lambda i: (0, 0)),
        ],
        out_specs=[
            pl.BlockSpec((_BM, HID), lambda i: (i, 0)),
            pl.BlockSpec((_BM, 2), lambda i: (i, 0)),
        ],
        out_shape=[
            jax.ShapeDtypeStruct((N, HID), jnp.float32),
            jax.ShapeDtypeStruct((N, 2), jnp.float32),
        ],
    )(x, w, a2)


def _combine_body(h0_ref, h1_ref, xl_ref, al2_ref, b_ref, c_ref, out_ref):
    h0 = h0_ref[...]
    h1 = h1_ref[...]
    fsum = jnp.concatenate([h0[:, :HW], h1[:, :HW]], axis=1)
    dsum = h0[:, 32:33]
    cnt = h0[:, 33:34]
    asum = h0[:, 34:35]
    la = asum / jnp.maximum(cnt, 1.0)
    al2 = al2_ref[...]
    alpha = al2[:, 0:1] + al2[:, 1:2] + c_ref[0, 0] * la
    alpha = jnp.where(alpha >= 0, alpha, 0.2 * alpha)
    exl = jnp.exp(alpha)
    xl = xl_ref[...]
    out = (fsum + exl * xl) / (dsum + exl + 1e-16) + b_ref[...]
    out_ref[...] = jnp.maximum(out, 0.0)


def _combine(h0, h1, xl, al2, b, c):
    return pl.pallas_call(
        _combine_body,
        grid=(N // _BM,),
        in_specs=[
            pl.BlockSpec((_BM, AW), lambda i: (i, 0)),
            pl.BlockSpec((_BM, AW), lambda i: (i, 0)),
            pl.BlockSpec((_BM, HID), lambda i: (i, 0)),
            pl.BlockSpec((_BM, 2), lambda i: (i, 0)),
            pl.BlockSpec((1, HID), lambda i: (0, 0)),
            pl.BlockSpec((1, 1), lambda i: (0, 0), memory_space=pltpu.SMEM),
        ],
        out_specs=pl.BlockSpec((_BM, HID), lambda i: (i, 0)),
        out_shape=jax.ShapeDtypeStruct((N, HID), jnp.float32),
    )(h0, h1, xl, al2, b[None, :], c)


_BG = 80


def _head_body(z_ref, w1_ref, b1_ref, w2_ref, b2_ref, out_ref):
    z1 = jnp.dot(z_ref[...], w1_ref[...], preferred_element_type=jnp.float32)
    z1 = jnp.maximum(z1 + b1_ref[...], 0.0)
    w2 = w2_ref[...]
    z20 = jnp.sum(z1 * w2[:, 0][None, :], axis=1, keepdims=True) + b2_ref[0, 0]
    z21 = jnp.sum(z1 * w2[:, 1][None, :], axis=1, keepdims=True) + b2_ref[0, 1]
    m = jnp.maximum(z20, z21)
    lse = m + jnp.log(jnp.exp(z20 - m) + jnp.exp(z21 - m))
    out_ref[...] = jnp.concatenate([z20 - lse, z21 - lse], axis=1)


def _head(z, w1, b1, w2, b2):
    dz = z.shape[1]
    return pl.pallas_call(
        _head_body,
        grid=(NG // _BG,),
        in_specs=[
            pl.BlockSpec((_BG, dz), lambda i: (i, 0)),
            pl.BlockSpec((dz, HID), lambda i: (0, 0)),
            pl.BlockSpec((1, HID), lambda i: (0, 0)),
            pl.BlockSpec((HID, 2), lambda i: (0, 0)),
            pl.BlockSpec((1, 2), lambda i: (0, 0)),
        ],
        out_specs=pl.BlockSpec((_BG, 2), lambda i: (i, 0)),
        out_shape=jax.ShapeDtypeStruct((NG, 2), jnp.float32),
    )(z, w1, b1[None, :], w2, b2[None, :])


def _gat_layer(x, src, dst, ea, zero, W, a_s, a_d, a_e, We, b):
    xl, al2 = _xform(x, W, a_s, a_d)
    c = jnp.dot(We[0], a_e).reshape(1, 1)
    halves = xl.reshape(N, 2, HW)
    alsrc = jnp.broadcast_to(al2[:, 0][:, None, None], (N, 2, 1))
    pad = jnp.zeros((N, 2, XW - HW - 1), jnp.float32)
    xl2 = jnp.concatenate([halves, alsrc, pad], axis=2).reshape(2 * N, XW)
    aldst2 = jnp.concatenate(
        [al2[:, 1][:, None], jnp.zeros((N, DW - 1), jnp.float32)], axis=1)
    cvec = jnp.broadcast_to(c.reshape(1), (L,))
    acc = _edge_kernel(xl2, src, dst, ea, aldst2, cvec, zero)
    return _combine(acc[0], acc[1], xl, al2, b, c)


def kernel(x, edge_index, batch, edge_attr, W1, a_src1, a_dst1, a_edge1, We1,
           b1, W2, a_src2, a_dst2, a_edge2, We2, b2, lin1_W, lin1_b, lin2_W,
           lin2_b):
    src = edge_index[0]
    dst = edge_index[1]
    ea = edge_attr[:, 0]
    zero = jnp.zeros((ROWS_PT, AW), jnp.float32)
    x1 = _gat_layer(x, src, dst, ea, zero, W1, a_src1, a_dst1, a_edge1, We1, b1)
    x2 = _gat_layer(x1, src, dst, ea, zero, W2, a_src2, a_dst2, a_edge2, We2, b2)
    h = jnp.concatenate([x1, x2], axis=1)
    z = h.reshape(NG, PG * 2 * HID)
    return _head(z, lin1_W, lin1_b, lin2_W, lin2_b)


# speedup vs baseline: 20.1884x; 1.6922x over previous
import functools

import jax
import jax.numpy as jnp
from jax import lax
from jax.experimental import pallas as pl
from jax.experimental.pallas import tpu as pltpu
from jax.experimental.pallas import tpu_sc as plsc

N = 36000
E = 576000
D_IN = 128
HID = 64
NG = 400
PG = 90

NC = 2
NS = 16
L = 16
NW = NC * NS

HW = HID // 2
AW = 48
XW = 48
DW = 16
EPT = E // NS
CH = 80
NCHUNK = EPT // CH
ROWS_PT = 2240
ROWS_TAIL = N - NS * ROWS_PT


def _edge_body(xl2_hbm, edges_hbm, aldst2_hbm, c_hbm,
               zero_hbm, acc_out,
               c_v, ebuf_v, idx_v, rows_v, adrows_v,
               buf2_v, acc_s, esem0, esem1, gsem0, gsem1):
    cid = lax.axis_index("c")
    sid = lax.axis_index("s")
    esem = (esem0, esem1)
    gsem = (gsem0, gsem1)

    pltpu.sync_copy(zero_hbm, acc_s.at[pl.ds(sid * ROWS_PT, ROWS_PT), :])
    @pl.when(sid == NS - 1)
    def _zero_tail():
        pltpu.sync_copy(zero_hbm.at[pl.ds(0, ROWS_TAIL), :],
                        acc_s.at[pl.ds(NS * ROWS_PT, ROWS_TAIL), :])

    pltpu.sync_copy(c_hbm, c_v)

    lanes = lax.iota(jnp.int32, L)
    zv = jnp.zeros((L,), jnp.float32)
    for i in range(CH):
        for j in range(AW // L):
            buf2_v[i, pl.ds(j * L, L)] = zv

    plsc.subcore_barrier()

    base = sid * EPT
    c32 = jnp.full((L,), 32, jnp.int32)
    c33 = jnp.full((L,), 33, jnp.int32)
    c34 = jnp.full((L,), 34, jnp.int32)
    z16 = jnp.zeros((L,), jnp.int32)

    def fire_edges(k, b):
        pltpu.async_copy(edges_hbm.at[:, pl.ds(base + k * CH, CH)],
                         ebuf_v.at[b], esem[b])

    def wait_edges(b):
        pltpu.make_async_copy(edges_hbm.at[:, pl.ds(0, CH)],
                              ebuf_v.at[b], esem[b]).wait()

    def build_idx_fire_gathers(k, b):
        for g in range(CH // L):
            s16 = ebuf_v[b, 0, pl.ds(g * L, L)]
            idx_v[b, pl.ds(g * L, L)] = s16 * 2 + cid
        pltpu.async_copy(xl2_hbm.at[idx_v.at[b]], rows_v.at[b], gsem[b])
        pltpu.async_copy(aldst2_hbm.at[ebuf_v.at[b, 1]], adrows_v.at[b], gsem[b])

    def wait_gathers(b):
        pltpu.make_async_copy(xl2_hbm.at[pl.ds(0, CH), :],
                              rows_v.at[b], gsem[b]).wait()
        pltpu.make_async_copy(aldst2_hbm.at[pl.ds(0, CH), :],
                              adrows_v.at[b], gsem[b]).wait()

    def process(b):
        cvec = c_v[pl.ds(0, L)]
        for g in range(CH // L):
            s16 = ebuf_v[b, 0, pl.ds(g * L, L)]
            d16 = ebuf_v[b, 1, pl.ds(g * L, L)]
            e16 = plsc.bitcast(ebuf_v[b, 2, pl.ds(g * L, L)], jnp.float32)
            validf = jnp.where(s16 != d16, 1.0, 0.0).astype(jnp.float32)
            rows16 = g * L + lanes
            plsc.store_scatter(buf2_v, [rows16, c33], validf)
            plsc.store_scatter(buf2_v, [rows16, c34], e16 * validf)
        wait_gathers(b)
        for g in range(CH // L):
            rows16 = g * L + lanes
            s16 = ebuf_v[b, 0, pl.ds(g * L, L)]
            d16 = ebuf_v[b, 1, pl.ds(g * L, L)]
            e16 = plsc.bitcast(ebuf_v[b, 2, pl.ds(g * L, L)], jnp.float32)
            asrc = plsc.load_gather(rows_v, [jnp.full((L,), b, jnp.int32), rows16, c32])
            adst = plsc.load_gather(adrows_v, [jnp.full((L,), b, jnp.int32), rows16, z16])
            alpha = asrc + adst + cvec * e16
            alpha = jnp.where(alpha >= 0, alpha, 0.2 * alpha)
            ex = jnp.exp(alpha) * jnp.where(s16 != d16, 1.0, 0.0)
            plsc.store_scatter(buf2_v, [rows16, c32], ex)
        def scale(i, c2):
            exs = buf2_v[i, pl.ds(2 * L, L)][0]
            buf2_v[i, pl.ds(0, L)] = rows_v[b, i, pl.ds(0, L)] * exs
            buf2_v[i, pl.ds(L, L)] = rows_v[b, i, pl.ds(L, L)] * exs
            return c2
        lax.fori_loop(0, CH, scale, 0, unroll=8)
        pltpu.sync_copy(buf2_v, acc_s.at[ebuf_v.at[b, 1]], add=True)

    pltpu.sync_copy(edges_hbm.at[:, pl.ds(base, CH)], ebuf_v.at[0])
    build_idx_fire_gathers(0, 0)
    fire_edges(1, 1)

    def pair(p, carry):
        wait_edges(1)
        build_idx_fire_gathers_k = build_idx_fire_gathers
        build_idx_fire_gathers_k(2 * p + 1, 1)
        process(0)
        @pl.when(p < NCHUNK // 2 - 1)
        def _fe0():
            fire_edges(2 * p + 2, 0)
        @pl.when(p < NCHUNK // 2 - 1)
        def _w0():
            wait_edges(0)
            build_idx_fire_gathers_k(2 * p + 2, 0)
        process(1)
        @pl.when(p < NCHUNK // 2 - 1)
        def _fe1():
            fire_edges(2 * p + 3, 1)
        return carry

    lax.fori_loop(0, NCHUNK // 2, pair, 0)
    plsc.subcore_barrier()
    pltpu.sync_copy(acc_s.at[pl.ds(sid * ROWS_PT, ROWS_PT), :],
                    acc_out.at[cid, pl.ds(sid * ROWS_PT, ROWS_PT), :])
    @pl.when(sid == NS - 1)
    def _flush_tail():
        pltpu.sync_copy(acc_s.at[pl.ds(NS * ROWS_PT, ROWS_TAIL), :],
                        acc_out.at[cid, pl.ds(NS * ROWS_PT, ROWS_TAIL), :])


_edge_kernel = functools.partial(
    pl.kernel,
    out_type=jax.ShapeDtypeStruct((NC, N, AW), jnp.float32),
    mesh=plsc.VectorSubcoreMesh(core_axis_name="c", subcore_axis_name="s",
                                num_cores=NC, num_subcores=NS),
    scratch_types=[
        pltpu.VMEM((L,), jnp.float32),
        pltpu.VMEM((2, 3, CH), jnp.int32),
        pltpu.VMEM((2, CH), jnp.int32),
        pltpu.VMEM((2, CH, XW), jnp.float32),
        pltpu.VMEM((2, CH, DW), jnp.float32),
        pltpu.VMEM((CH, AW), jnp.float32),
        pltpu.VMEM_SHARED((N, AW), jnp.float32),
        pltpu.SemaphoreType.DMA,
        pltpu.SemaphoreType.DMA,
        pltpu.SemaphoreType.DMA,
        pltpu.SemaphoreType.DMA,
    ],
    compiler_params=pltpu.CompilerParams(needs_layout_passes=False,
                                         use_tc_tiling_on_sc=False),
)(_edge_body)


_BM = 1440


def _xform_body(x_ref, w_ref, a2_ref, xl_ref, al2_ref):
    xl = jnp.dot(x_ref[...], w_ref[...], preferred_element_type=jnp.float32)
    xl_ref[...] = xl
    al2_ref[...] = jnp.dot(xl, a2_ref[...], preferred_element_type=jnp.float32)


def _xform(x, w, a_s, a_d):
    din = x.shape[1]
    a2 = jnp.stack([a_s, a_d], axis=1)
    return pl.pallas_call(
        _xform_body,
        grid=(N // _BM,),
        in_specs=[
            pl.BlockSpec((_BM, din), lambda i: (i, 0)),
            pl.BlockSpec((din, HID), lambda i: (0, 0)),
            pl.BlockSpec((HID, 2), lambda i: (0, 0)),
        ],
        out_specs=[
            pl.BlockSpec((_BM, HID), lambda i: (i, 0)),
            pl.BlockSpec((_BM, 2), lambda i: (i, 0)),
        ],
        out_shape=[
            jax.ShapeDtypeStruct((N, HID), jnp.float32),
            jax.ShapeDtypeStruct((N, 2), jnp.float32),
        ],
    )(x, w, a2)


def _combine_body(h0_ref, h1_ref, xl_ref, al2_ref, b_ref, c_ref, out_ref):
    h0 = h0_ref[...]
    h1 = h1_ref[...]
    fsum = jnp.concatenate([h0[:, :HW], h1[:, :HW]], axis=1)
    dsum = h0[:, 32:33]
    cnt = h0[:, 33:34]
    asum = h0[:, 34:35]
    la = asum / jnp.maximum(cnt, 1.0)
    al2 = al2_ref[...]
    alpha = al2[:, 0:1] + al2[:, 1:2] + c_ref[0, 0] * la
    alpha = jnp.where(alpha >= 0, alpha, 0.2 * alpha)
    exl = jnp.exp(alpha)
    xl = xl_ref[...]
    out = (fsum + exl * xl) / (dsum + exl + 1e-16) + b_ref[...]
    out_ref[...] = jnp.maximum(out, 0.0)


def _combine(h0, h1, xl, al2, b, c):
    return pl.pallas_call(
        _combine_body,
        grid=(N // _BM,),
        in_specs=[
            pl.BlockSpec((_BM, AW), lambda i: (i, 0)),
            pl.BlockSpec((_BM, AW), lambda i: (i, 0)),
            pl.BlockSpec((_BM, HID), lambda i: (i, 0)),
            pl.BlockSpec((_BM, 2), lambda i: (i, 0)),
            pl.BlockSpec((1, HID), lambda i: (0, 0)),
            pl.BlockSpec((1, 1), lambda i: (0, 0), memory_space=pltpu.SMEM),
        ],
        out_specs=pl.BlockSpec((_BM, HID), lambda i: (i, 0)),
        out_shape=jax.ShapeDtypeStruct((N, HID), jnp.float32),
    )(h0, h1, xl, al2, b[None, :], c)


_BG = 80


def _head_body(z_ref, w1_ref, b1_ref, w2_ref, b2_ref, out_ref):
    z1 = jnp.dot(z_ref[...], w1_ref[...], preferred_element_type=jnp.float32)
    z1 = jnp.maximum(z1 + b1_ref[...], 0.0)
    w2 = w2_ref[...]
    z20 = jnp.sum(z1 * w2[:, 0][None, :], axis=1, keepdims=True) + b2_ref[0, 0]
    z21 = jnp.sum(z1 * w2[:, 1][None, :], axis=1, keepdims=True) + b2_ref[0, 1]
    m = jnp.maximum(z20, z21)
    lse = m + jnp.log(jnp.exp(z20 - m) + jnp.exp(z21 - m))
    out_ref[...] = jnp.concatenate([z20 - lse, z21 - lse], axis=1)


def _head(z, w1, b1, w2, b2):
    dz = z.shape[1]
    return pl.pallas_call(
        _head_body,
        grid=(NG // _BG,),
        in_specs=[
            pl.BlockSpec((_BG, dz), lambda i: (i, 0)),
            pl.BlockSpec((dz, HID), lambda i: (0, 0)),
            pl.BlockSpec((1, HID), lambda i: (0, 0)),
            pl.BlockSpec((HID, 2), lambda i: (0, 0)),
            pl.BlockSpec((1, 2), lambda i: (0, 0)),
        ],
        out_specs=pl.BlockSpec((_BG, 2), lambda i: (i, 0)),
        out_shape=jax.ShapeDtypeStruct((NG, 2), jnp.float32),
    )(z, w1, b1[None, :], w2, b2[None, :])


def _gat_layer(x, edges, zero, W, a_s, a_d, a_e, We, b):
    xl, al2 = _xform(x, W, a_s, a_d)
    c = jnp.dot(We[0], a_e).reshape(1, 1)
    halves = xl.reshape(N, 2, HW)
    alsrc = jnp.broadcast_to(al2[:, 0][:, None, None], (N, 2, 1))
    pad = jnp.zeros((N, 2, XW - HW - 1), jnp.float32)
    xl2 = jnp.concatenate([halves, alsrc, pad], axis=2).reshape(2 * N, XW)
    aldst2 = jnp.concatenate(
        [al2[:, 1][:, None], jnp.zeros((N, DW - 1), jnp.float32)], axis=1)
    cvec = jnp.broadcast_to(c.reshape(1), (L,))
    acc = _edge_kernel(xl2, edges, aldst2, cvec, zero)
    return _combine(acc[0], acc[1], xl, al2, b, c)


def kernel(x, edge_index, batch, edge_attr, W1, a_src1, a_dst1, a_edge1, We1,
           b1, W2, a_src2, a_dst2, a_edge2, We2, b2, lin1_W, lin1_b, lin2_W,
           lin2_b):
    edges = jnp.stack([edge_index[0], edge_index[1],
                       edge_attr[:, 0].view(jnp.int32)], axis=0)
    zero = jnp.zeros((ROWS_PT, AW), jnp.float32)
    x1 = _gat_layer(x, edges, zero, W1, a_src1, a_dst1, a_edge1, We1, b1)
    x2 = _gat_layer(x1, edges, zero, W2, a_src2, a_dst2, a_edge2, We2, b2)
    h = jnp.concatenate([x1, x2], axis=1)
    z = h.reshape(NG, PG * 2 * HID)
    return _head(z, lin1_W, lin1_b, lin2_W, lin2_b)


# speedup vs baseline: 23.5640x; 1.1672x over previous
import functools

import jax
import jax.numpy as jnp
from jax import lax
from jax.experimental import pallas as pl
from jax.experimental.pallas import tpu as pltpu
from jax.experimental.pallas import tpu_sc as plsc

N = 36000
E = 576000
D_IN = 128
HID = 64
NG = 400
PG = 90

NC = 2
NS = 16
L = 16
NW = NC * NS

HW = HID // 2
AW = 48
XW = HW
DW = 16
EPT = E // NS
CH = 80
NCHUNK = EPT // CH
ROWS_PT = 2240
ROWS_TAIL = N - NS * ROWS_PT


def _edge_body(xl2_hbm, edges_hbm, alsd_hbm, c_hbm,
               zero_hbm, acc_out,
               c_v, ebuf_v, idx_v, rows_v, asrows_v, adrows_v,
               buf2_v, acc_s, esem0, esem1, gsem0, gsem1):
    cid = lax.axis_index("c")
    sid = lax.axis_index("s")
    esem = (esem0, esem1)
    gsem = (gsem0, gsem1)

    pltpu.sync_copy(zero_hbm, acc_s.at[pl.ds(sid * ROWS_PT, ROWS_PT), :])
    @pl.when(sid == NS - 1)
    def _zero_tail():
        pltpu.sync_copy(zero_hbm.at[pl.ds(0, ROWS_TAIL), :],
                        acc_s.at[pl.ds(NS * ROWS_PT, ROWS_TAIL), :])

    pltpu.sync_copy(c_hbm, c_v)

    lanes = lax.iota(jnp.int32, L)
    zv = jnp.zeros((L,), jnp.float32)
    for i in range(CH):
        for j in range(AW // L):
            buf2_v[i, pl.ds(j * L, L)] = zv

    plsc.subcore_barrier()

    base = sid * EPT
    c32 = jnp.full((L,), 32, jnp.int32)
    c33 = jnp.full((L,), 33, jnp.int32)
    c34 = jnp.full((L,), 34, jnp.int32)
    z16 = jnp.zeros((L,), jnp.int32)

    def fire_edges(k, b):
        pltpu.async_copy(edges_hbm.at[:, pl.ds(base + k * CH, CH)],
                         ebuf_v.at[b], esem[b])

    def wait_edges(b):
        pltpu.make_async_copy(edges_hbm.at[:, pl.ds(0, CH)],
                              ebuf_v.at[b], esem[b]).wait()

    def build_idx_fire_gathers(k, b):
        for g in range(CH // L):
            s16 = ebuf_v[b, 0, pl.ds(g * L, L)]
            idx_v[b, pl.ds(g * L, L)] = s16 * 2 + cid
        pltpu.async_copy(xl2_hbm.at[idx_v.at[b]], rows_v.at[b], gsem[b])
        pltpu.async_copy(alsd_hbm.at[ebuf_v.at[b, 0]], asrows_v.at[b], gsem[b])
        pltpu.async_copy(alsd_hbm.at[ebuf_v.at[b, 1]], adrows_v.at[b], gsem[b])

    def wait_gathers(b):
        pltpu.make_async_copy(xl2_hbm.at[pl.ds(0, CH), :],
                              rows_v.at[b], gsem[b]).wait()
        pltpu.make_async_copy(alsd_hbm.at[pl.ds(0, CH), :],
                              asrows_v.at[b], gsem[b]).wait()
        pltpu.make_async_copy(alsd_hbm.at[pl.ds(0, CH), :],
                              adrows_v.at[b], gsem[b]).wait()

    def process(b):
        cvec = c_v[pl.ds(0, L)]
        for g in range(CH // L):
            s16 = ebuf_v[b, 0, pl.ds(g * L, L)]
            d16 = ebuf_v[b, 1, pl.ds(g * L, L)]
            e16 = plsc.bitcast(ebuf_v[b, 2, pl.ds(g * L, L)], jnp.float32)
            validf = jnp.where(s16 != d16, 1.0, 0.0).astype(jnp.float32)
            rows16 = g * L + lanes
            plsc.store_scatter(buf2_v, [rows16, c33], validf)
            plsc.store_scatter(buf2_v, [rows16, c34], e16 * validf)
        wait_gathers(b)
        for g in range(CH // L):
            rows16 = g * L + lanes
            s16 = ebuf_v[b, 0, pl.ds(g * L, L)]
            d16 = ebuf_v[b, 1, pl.ds(g * L, L)]
            e16 = plsc.bitcast(ebuf_v[b, 2, pl.ds(g * L, L)], jnp.float32)
            bvec = jnp.full((L,), b, jnp.int32)
            asrc = plsc.load_gather(asrows_v, [bvec, rows16, z16])
            adst = plsc.load_gather(adrows_v, [bvec, rows16, jnp.full((L,), 1, jnp.int32)])
            alpha = asrc + adst + cvec * e16
            alpha = jnp.where(alpha >= 0, alpha, 0.2 * alpha)
            ex = jnp.exp(alpha) * jnp.where(s16 != d16, 1.0, 0.0)
            plsc.store_scatter(buf2_v, [rows16, c32], ex)
        def scale(i, c2):
            exs = buf2_v[i, pl.ds(2 * L, L)][0]
            buf2_v[i, pl.ds(0, L)] = rows_v[b, i, pl.ds(0, L)] * exs
            buf2_v[i, pl.ds(L, L)] = rows_v[b, i, pl.ds(L, L)] * exs
            return c2
        lax.fori_loop(0, CH, scale, 0, unroll=8)
        pltpu.sync_copy(buf2_v, acc_s.at[ebuf_v.at[b, 1]], add=True)

    pltpu.sync_copy(edges_hbm.at[:, pl.ds(base, CH)], ebuf_v.at[0])
    build_idx_fire_gathers(0, 0)
    fire_edges(1, 1)

    def pair(p, carry):
        wait_edges(1)
        build_idx_fire_gathers_k = build_idx_fire_gathers
        build_idx_fire_gathers_k(2 * p + 1, 1)
        process(0)
        @pl.when(p < NCHUNK // 2 - 1)
        def _fe0():
            fire_edges(2 * p + 2, 0)
        @pl.when(p < NCHUNK // 2 - 1)
        def _w0():
            wait_edges(0)
            build_idx_fire_gathers_k(2 * p + 2, 0)
        process(1)
        @pl.when(p < NCHUNK // 2 - 1)
        def _fe1():
            fire_edges(2 * p + 3, 1)
        return carry

    lax.fori_loop(0, NCHUNK // 2, pair, 0)
    plsc.subcore_barrier()
    pltpu.sync_copy(acc_s.at[pl.ds(sid * ROWS_PT, ROWS_PT), :],
                    acc_out.at[cid, pl.ds(sid * ROWS_PT, ROWS_PT), :])
    @pl.when(sid == NS - 1)
    def _flush_tail():
        pltpu.sync_copy(acc_s.at[pl.ds(NS * ROWS_PT, ROWS_TAIL), :],
                        acc_out.at[cid, pl.ds(NS * ROWS_PT, ROWS_TAIL), :])


_edge_kernel = functools.partial(
    pl.kernel,
    out_type=jax.ShapeDtypeStruct((NC, N, AW), jnp.float32),
    mesh=plsc.VectorSubcoreMesh(core_axis_name="c", subcore_axis_name="s",
                                num_cores=NC, num_subcores=NS),
    scratch_types=[
        pltpu.VMEM((L,), jnp.float32),
        pltpu.VMEM((2, 3, CH), jnp.int32),
        pltpu.VMEM((2, CH), jnp.int32),
        pltpu.VMEM((2, CH, XW), jnp.float32),
        pltpu.VMEM((2, CH, DW), jnp.float32),
        pltpu.VMEM((2, CH, DW), jnp.float32),
        pltpu.VMEM((CH, AW), jnp.float32),
        pltpu.VMEM_SHARED((N, AW), jnp.float32),
        pltpu.SemaphoreType.DMA,
        pltpu.SemaphoreType.DMA,
        pltpu.SemaphoreType.DMA,
        pltpu.SemaphoreType.DMA,
    ],
    compiler_params=pltpu.CompilerParams(needs_layout_passes=False,
                                         use_tc_tiling_on_sc=False),
)(_edge_body)


_BM = 1440


def _xform_body(x_ref, w_ref, a2_ref, xl_ref, al2_ref):
    xl = jnp.dot(x_ref[...], w_ref[...], preferred_element_type=jnp.float32)
    xl_ref[...] = xl
    al2_ref[...] = jnp.dot(xl, a2_ref[...], preferred_element_type=jnp.float32)


def _xform(x, w, a_s, a_d):
    din = x.shape[1]
    a2 = jnp.stack(
        [a_s, a_d] + [jnp.zeros((HID,), jnp.float32)] * (DW - 2), axis=1)
    return pl.pallas_call(
        _xform_body,
        grid=(N // _BM,),
        in_specs=[
            pl.BlockSpec((_BM, din), lambda i: (i, 0)),
            pl.BlockSpec((din, HID), lambda i: (0, 0)),
            pl.BlockSpec((HID, DW), lambda i: (0, 0)),
        ],
        out_specs=[
            pl.BlockSpec((_BM, HID), lambda i: (i, 0)),
            pl.BlockSpec((_BM, DW), lambda i: (i, 0)),
        ],
        out_shape=[
            jax.ShapeDtypeStruct((N, HID), jnp.float32),
            jax.ShapeDtypeStruct((N, DW), jnp.float32),
        ],
    )(x, w, a2)


def _combine_body(h0_ref, h1_ref, xl_ref, al2_ref, b_ref, c_ref, out_ref):
    h0 = h0_ref[...]
    h1 = h1_ref[...]
    fsum = jnp.concatenate([h0[:, :HW], h1[:, :HW]], axis=1)
    dsum = h0[:, 32:33]
    cnt = h0[:, 33:34]
    asum = h0[:, 34:35]
    la = asum / jnp.maximum(cnt, 1.0)
    al2 = al2_ref[...]
    alpha = al2[:, 0:1] + al2[:, 1:2] + c_ref[0, 0] * la
    alpha = jnp.where(alpha >= 0, alpha, 0.2 * alpha)
    exl = jnp.exp(alpha)
    xl = xl_ref[...]
    out = (fsum + exl * xl) / (dsum + exl + 1e-16) + b_ref[...]
    out_ref[...] = jnp.maximum(out, 0.0)


def _combine(h0, h1, xl, al2, b, c):
    return pl.pallas_call(
        _combine_body,
        grid=(N // _BM,),
        in_specs=[
            pl.BlockSpec((_BM, AW), lambda i: (i, 0)),
            pl.BlockSpec((_BM, AW), lambda i: (i, 0)),
            pl.BlockSpec((_BM, HID), lambda i: (i, 0)),
            pl.BlockSpec((_BM, DW), lambda i: (i, 0)),
            pl.BlockSpec((1, HID), lambda i: (0, 0)),
            pl.BlockSpec((1, 1), lambda i: (0, 0), memory_space=pltpu.SMEM),
        ],
        out_specs=pl.BlockSpec((_BM, HID), lambda i: (i, 0)),
        out_shape=jax.ShapeDtypeStruct((N, HID), jnp.float32),
    )(h0, h1, xl, al2, b[None, :], c)


_BG = 80


def _head_body(z_ref, w1_ref, b1_ref, w2_ref, b2_ref, out_ref):
    z1 = jnp.dot(z_ref[...], w1_ref[...], preferred_element_type=jnp.float32)
    z1 = jnp.maximum(z1 + b1_ref[...], 0.0)
    w2 = w2_ref[...]
    z20 = jnp.sum(z1 * w2[:, 0][None, :], axis=1, keepdims=True) + b2_ref[0, 0]
    z21 = jnp.sum(z1 * w2[:, 1][None, :], axis=1, keepdims=True) + b2_ref[0, 1]
    m = jnp.maximum(z20, z21)
    lse = m + jnp.log(jnp.exp(z20 - m) + jnp.exp(z21 - m))
    out_ref[...] = jnp.concatenate([z20 - lse, z21 - lse], axis=1)


def _head(z, w1, b1, w2, b2):
    dz = z.shape[1]
    return pl.pallas_call(
        _head_body,
        grid=(NG // _BG,),
        in_specs=[
            pl.BlockSpec((_BG, dz), lambda i: (i, 0)),
            pl.BlockSpec((dz, HID), lambda i: (0, 0)),
            pl.BlockSpec((1, HID), lambda i: (0, 0)),
            pl.BlockSpec((HID, 2), lambda i: (0, 0)),
            pl.BlockSpec((1, 2), lambda i: (0, 0)),
        ],
        out_specs=pl.BlockSpec((_BG, 2), lambda i: (i, 0)),
        out_shape=jax.ShapeDtypeStruct((NG, 2), jnp.float32),
    )(z, w1, b1[None, :], w2, b2[None, :])


def _gat_layer(x, edges, zero, W, a_s, a_d, a_e, We, b):
    xl, al2p = _xform(x, W, a_s, a_d)
    c = jnp.dot(We[0], a_e).reshape(1, 1)
    xl2 = xl.reshape(2 * N, XW)
    cvec = jnp.broadcast_to(c.reshape(1), (L,))
    acc = _edge_kernel(xl2, edges, al2p, cvec, zero)
    return _combine(acc[0], acc[1], xl, al2p, b, c)


def kernel(x, edge_index, batch, edge_attr, W1, a_src1, a_dst1, a_edge1, We1,
           b1, W2, a_src2, a_dst2, a_edge2, We2, b2, lin1_W, lin1_b, lin2_W,
           lin2_b):
    edges = jnp.stack([edge_index[0], edge_index[1],
                       edge_attr[:, 0].view(jnp.int32)], axis=0)
    zero = jnp.zeros((ROWS_PT, AW), jnp.float32)
    x1 = _gat_layer(x, edges, zero, W1, a_src1, a_dst1, a_edge1, We1, b1)
    x2 = _gat_layer(x1, edges, zero, W2, a_src2, a_dst2, a_edge2, We2, b2)
    h = jnp.concatenate([x1, x2], axis=1)
    z = h.reshape(NG, PG * 2 * HID)
    return _head(z, lin1_W, lin1_b, lin2_W, lin2_b)
